# Initial kernel scaffold; baseline (speedup 1.0000x reference)
import jax
import jax.numpy as jnp
from jax.experimental import pallas as pl


def kernel(features, edge_index, weight, edge_weight, W1, b1, W3, b3):
    raise NotImplementedError("write your pallas kernel here")


# speedup vs baseline: 3.6176x; 3.6176x over previous
import functools

import jax
import jax.numpy as jnp
from jax import lax
from jax.experimental import pallas as pl
from jax.experimental.pallas import tpu as pltpu
from jax.experimental.pallas import tpu_sc as plsc

_N = 10000
_E = 320000
_D = 128
_HID = 256
_ND = 128
_NC = 64

_B = 128
_E_PAD = 323584
_PAD = _E_PAD - _E
_N_SUB = 16
_N_PAD = 10112
_ROWS_PT = _N_PAD // _N_SUB
_R = 1000

_MESH = plsc.VectorSubcoreMesh(core_axis_name="c", subcore_axis_name="s")


@functools.partial(
    pl.kernel,
    out_type=jax.ShapeDtypeStruct((2, _N_PAD, 128), jnp.float32),
    mesh=_MESH,
    scratch_types=[
        pltpu.VMEM((_B,), jnp.int32),
        pltpu.VMEM((_B, 128), jnp.float32),
        pltpu.VMEM_SHARED((_N_PAD, 128), jnp.float32),
    ],
)
def _degree_hist(idx_hbm, ones_hbm, zeros_hbm, out_hbm, idxb, onesb, accum):
    c = lax.axis_index("c")
    s = lax.axis_index("s")
    rows = pl.ds(s * _ROWS_PT, _ROWS_PT)
    pltpu.sync_copy(ones_hbm, onesb)
    pltpu.sync_copy(zeros_hbm.at[rows], accum.at[rows])
    plsc.subcore_barrier()
    cpt = _E_PAD // _N_SUB // _B
    def _chunk(g, carry):
        base = c * _E_PAD + (s * cpt + g) * _B
        pltpu.sync_copy(idx_hbm.at[pl.ds(base, _B)], idxb)
        pltpu.sync_copy(onesb, accum.at[idxb], add=True)
        return carry
    lax.fori_loop(0, cpt, _chunk, 0)
    plsc.subcore_barrier()
    pltpu.sync_copy(accum.at[rows], out_hbm.at[c, rows])


def _make_edge_pass(n_tables, split_edges):
    n_workers = _N_SUB * (2 if split_edges else 1)
    cpt = _E_PAD // n_workers // _B
    tshape = (2, _N, 128) if n_tables == 2 else (_N, 128)

    @functools.partial(
        pl.kernel,
        out_type=jax.ShapeDtypeStruct((2, _N_PAD, 128), jnp.float32),
        mesh=_MESH,
        scratch_types=[
            pltpu.VMEM((_B,), jnp.int32),
            pltpu.VMEM((_B,), jnp.int32),
            pltpu.VMEM((_B, 16), jnp.float32),
            pltpu.VMEM((_B, 128), jnp.float32),
            pltpu.VMEM_SHARED((_N_PAD, 128), jnp.float32),
            pltpu.SemaphoreType.DMA,
        ],
    )
    def _edge_pass(table_hbm, src_hbm, dst_hbm, ew_hbm, zeros_hbm, out_hbm,
                   srcb, dstb, ewb, gbuf, accum, sem):
        c = lax.axis_index("c")
        s = lax.axis_index("s")
        rows = pl.ds(s * _ROWS_PT, _ROWS_PT)
        pltpu.sync_copy(zeros_hbm.at[rows], accum.at[rows])
        plsc.subcore_barrier()
        tbl = table_hbm.at[c] if n_tables == 2 else table_hbm
        wid = (s * 2 + c) if split_edges else s

        def _chunk(g, carry):
            base = (wid * cpt + g) * _B
            pltpu.sync_copy(src_hbm.at[pl.ds(base, _B)], srcb)
            pltpu.sync_copy(dst_hbm.at[pl.ds(base, _B)], dstb)
            pltpu.sync_copy(ew_hbm.at[pl.ds(base, _B)], ewb)
            pltpu.async_copy(tbl.at[srcb], gbuf, sem).wait()

            def _scale(e, cc):
                w = ewb[e, :]
                for j in range(8):
                    sl = pl.ds(j * 16, 16)
                    gbuf[e, sl] = gbuf[e, sl] * w
                return cc
            lax.fori_loop(0, _B, _scale, 0)
            pltpu.sync_copy(gbuf, accum.at[dstb], add=True)
            return carry
        lax.fori_loop(0, cpt, _chunk, 0)
        plsc.subcore_barrier()
        pltpu.sync_copy(accum.at[rows], out_hbm.at[c, rows])

    def _call(table, src, dst, ew, zeros):
        assert table.shape == tshape
        return _edge_pass(table, src, dst, ew, zeros)
    return _call


_edge_pass_split_feat = _make_edge_pass(n_tables=2, split_edges=False)
_edge_pass_split_edge = _make_edge_pass(n_tables=1, split_edges=True)


def _row_scale(cnt_col, i):
    row = lax.broadcasted_iota(jnp.int32, (_R,), 0) + i * _R
    deg = cnt_col - jnp.where(row == 0, jnp.float32(_PAD), jnp.float32(0.0))
    return lax.rsqrt(jnp.maximum(deg, 1.0))


def _dense1_body(x_ref, w_ref, cnt_ref, o_ref):
    i = pl.program_id(1)
    r_out = _row_scale(cnt_ref[0, :, 0], i)
    o_ref[0, :, :] = jnp.dot(x_ref[...], w_ref[...],
                             preferred_element_type=jnp.float32) * r_out[:, None]


def _dense1(x, w1, cnt):
    return pl.pallas_call(
        _dense1_body,
        grid=(2, _N // _R),
        in_specs=[
            pl.BlockSpec((_R, _D), lambda h, i: (i, 0)),
            pl.BlockSpec((_D, 128), lambda h, i: (0, h)),
            pl.BlockSpec((1, _R, 128), lambda h, i: (0, i, 0)),
        ],
        out_specs=pl.BlockSpec((1, _R, 128), lambda h, i: (h, i, 0)),
        out_shape=jax.ShapeDtypeStruct((2, _N, 128), jnp.float32),
    )(x, w1, cnt)


def _dense2_body(u_ref, cnt_ref, b1_ref, w2_ref, o_ref):
    i = pl.program_id(0)
    r_in = _row_scale(cnt_ref[1, :, 0], i)
    r_out = _row_scale(cnt_ref[0, :, 0], i)
    b1 = b1_ref[...]
    x1a = u_ref[0] * r_in[:, None] + b1[:, :128]
    x1b = u_ref[1] * r_in[:, None] + b1[:, 128:]
    h2 = (jnp.dot(x1a, w2_ref[:128, :], preferred_element_type=jnp.float32)
          + jnp.dot(x1b, w2_ref[128:, :], preferred_element_type=jnp.float32))
    o_ref[...] = h2 * r_out[:, None]


def _dense2(u1, cnt, b1, w2):
    return pl.pallas_call(
        _dense2_body,
        grid=(_N // _R,),
        in_specs=[
            pl.BlockSpec((2, _R, 128), lambda i: (0, i, 0)),
            pl.BlockSpec((2, _R, 128), lambda i: (0, i, 0)),
            pl.BlockSpec((1, _HID), lambda i: (0, 0)),
            pl.BlockSpec((_HID, _ND), lambda i: (0, 0)),
        ],
        out_specs=pl.BlockSpec((_R, _ND), lambda i: (i, 0)),
        out_shape=jax.ShapeDtypeStruct((_N, _ND), jnp.float32),
    )(u1, cnt, b1, w2)


def _dense3_body(u_ref, cnt_ref, w3_ref, b3_ref, o_ref):
    i = pl.program_id(0)
    r_in = _row_scale(cnt_ref[1, :, 0], i)
    x = jnp.maximum((u_ref[0] + u_ref[1]) * r_in[:, None], 0.0)
    logits = jnp.dot(x, w3_ref[...], preferred_element_type=jnp.float32) + b3_ref[...]
    m = jnp.max(logits, axis=1, keepdims=True)
    ex = jnp.exp(logits - m)
    o_ref[...] = ex / jnp.sum(ex, axis=1, keepdims=True)


def _dense3(u2, cnt, w3, b3):
    return pl.pallas_call(
        _dense3_body,
        grid=(_N // _R,),
        in_specs=[
            pl.BlockSpec((2, _R, 128), lambda i: (0, i, 0)),
            pl.BlockSpec((2, _R, 128), lambda i: (0, i, 0)),
            pl.BlockSpec((_ND, _NC), lambda i: (0, 0)),
            pl.BlockSpec((1, _NC), lambda i: (0, 0)),
        ],
        out_specs=pl.BlockSpec((_R, _NC), lambda i: (i, 0)),
        out_shape=jax.ShapeDtypeStruct((_N, _NC), jnp.float32),
    )(u2, cnt, w3, b3)


def kernel(features, edge_index, weight, edge_weight, W1, b1, W3, b3):
    idx2 = jnp.pad(edge_index, ((0, 0), (0, _PAD)))
    idx_flat = idx2.reshape(2 * _E_PAD)
    ewp = jnp.broadcast_to(jnp.pad(edge_weight, (0, _PAD))[:, None],
                           (_E_PAD, 16)).astype(jnp.float32)
    ones128 = jnp.ones((_B, 128), jnp.float32)
    z128 = jnp.zeros((_N_PAD, 128), jnp.float32)

    cnt = _degree_hist(idx_flat, ones128, z128)
    h1s = _dense1(features, W1, cnt)
    u1 = _edge_pass_split_feat(h1s, idx2[0], idx2[1], ewp, z128)
    h2s = _dense2(u1, cnt, b1.reshape(1, _HID), weight)
    u2 = _edge_pass_split_edge(h2s, idx2[0], idx2[1], ewp, z128)
    return _dense3(u2, cnt, W3, b3.reshape(1, _NC))


# speedup vs baseline: 3.8028x; 1.0512x over previous
import functools

import jax
import jax.numpy as jnp
from jax import lax
from jax.experimental import pallas as pl
from jax.experimental.pallas import tpu as pltpu
from jax.experimental.pallas import tpu_sc as plsc

_N = 10000
_E = 320000
_D = 128
_HID = 256
_ND = 128
_NC = 64

_B = 128
_E_PAD = 327680
_PAD = _E_PAD - _E
_N_SUB = 16
_N_PAD = 10112
_ROWS_PT = _N_PAD // _N_SUB
_R = 1000

_MESH = plsc.VectorSubcoreMesh(core_axis_name="c", subcore_axis_name="s")


@functools.partial(
    pl.kernel,
    out_type=jax.ShapeDtypeStruct((2, _N_PAD, 128), jnp.float32),
    mesh=_MESH,
    scratch_types=[
        pltpu.VMEM((_B,), jnp.int32),
        pltpu.VMEM((_B, 128), jnp.float32),
        pltpu.VMEM_SHARED((_N_PAD, 128), jnp.float32),
    ],
)
def _degree_hist(idx_hbm, ones_hbm, zeros_hbm, out_hbm, idxb, onesb, accum):
    c = lax.axis_index("c")
    s = lax.axis_index("s")
    rows = pl.ds(s * _ROWS_PT, _ROWS_PT)
    pltpu.sync_copy(ones_hbm, onesb)
    pltpu.sync_copy(zeros_hbm.at[rows], accum.at[rows])
    plsc.subcore_barrier()
    cpt = _E_PAD // _N_SUB // _B
    def _chunk(g, carry):
        base = c * _E_PAD + (s * cpt + g) * _B
        pltpu.sync_copy(idx_hbm.at[pl.ds(base, _B)], idxb)
        pltpu.sync_copy(onesb, accum.at[idxb], add=True)
        return carry
    lax.fori_loop(0, cpt, _chunk, 0)
    plsc.subcore_barrier()
    pltpu.sync_copy(accum.at[rows], out_hbm.at[c, rows])


def _make_edge_pass(n_tables, split_edges):
    n_workers = _N_SUB * (2 if split_edges else 1)
    bg = 64
    cpt = _E_PAD // n_workers // bg
    tshape = (2, _N, 128) if n_tables == 2 else (_N, 128)

    @functools.partial(
        pl.kernel,
        out_type=jax.ShapeDtypeStruct((2, _N_PAD, 128), jnp.float32),
        mesh=_MESH,
        scratch_types=[
            pltpu.VMEM((2, bg), jnp.int32),
            pltpu.VMEM((2, bg), jnp.int32),
            pltpu.VMEM((2, bg), jnp.int32),
            pltpu.VMEM((2, bg, 16), jnp.float32),
            pltpu.VMEM((2, bg, 128), jnp.float32),
            pltpu.VMEM_SHARED((_N_PAD, 128), jnp.float32),
            pltpu.SemaphoreType.DMA,
            pltpu.SemaphoreType.DMA,
            pltpu.SemaphoreType.DMA,
            pltpu.SemaphoreType.DMA,
            pltpu.SemaphoreType.DMA,
            pltpu.SemaphoreType.DMA,
        ],
    )
    def _edge_pass(table_hbm, src_hbm, dst_hbm, ew_hbm, zeros_hbm, out_hbm,
                   srcb, dstb, dsts, ewb, gbuf, accum,
                   si0, si1, sg0, sg1, ss0, ss1):
        c = lax.axis_index("c")
        s = lax.axis_index("s")
        rows = pl.ds(s * _ROWS_PT, _ROWS_PT)
        pltpu.sync_copy(zeros_hbm.at[rows], accum.at[rows])
        plsc.subcore_barrier()
        tbl = table_hbm.at[c] if n_tables == 2 else table_hbm
        wid = (s * 2 + c) if split_edges else s
        semi, semg, sems = (si0, si1), (sg0, sg1), (ss0, ss1)

        def idx_start(g, b):
            base = (wid * cpt + g) * bg
            pltpu.async_copy(src_hbm.at[pl.ds(base, bg)], srcb.at[b], semi[b])
            pltpu.async_copy(dst_hbm.at[pl.ds(base, bg)], dstb.at[b], semi[b])
            pltpu.async_copy(ew_hbm.at[pl.ds(base, bg)], ewb.at[b], semi[b])

        def idx_wait(b):
            pltpu.make_async_copy(src_hbm.at[pl.ds(0, bg)], srcb.at[b], semi[b]).wait()
            pltpu.make_async_copy(dst_hbm.at[pl.ds(0, bg)], dstb.at[b], semi[b]).wait()
            pltpu.make_async_copy(ew_hbm.at[pl.ds(0, bg)], ewb.at[b], semi[b]).wait()

        def gather_start(b):
            pltpu.async_copy(tbl.at[srcb.at[b]], gbuf.at[b], semg[b])

        def gather_wait(b):
            pltpu.make_async_copy(tbl.at[srcb.at[b]], gbuf.at[b], semg[b]).wait()

        def scat_start(b):
            pltpu.async_copy(gbuf.at[b], accum.at[dsts.at[b]], sems[b], add=True)

        def scat_wait(b):
            pltpu.make_async_copy(gbuf.at[b], accum.at[dsts.at[b]], sems[b]).wait()

        idx_start(0, 0)
        idx_wait(0)
        gather_start(0)
        idx_start(1, 1)

        def _pair(h, carry):
            for b in (0, 1):
                g = h * 2 + b
                nb = 1 - b
                @pl.when((g >= 1) & (g + 1 < cpt))
                def _():
                    scat_wait(nb)
                @pl.when(g + 1 < cpt)
                def _():
                    idx_wait(nb)
                    gather_start(nb)
                gather_wait(b)
                eb, gb = ewb.at[b], gbuf.at[b]
                def _scale(e, cc):
                    w = eb[e, :]
                    for j in range(8):
                        sl = pl.ds(j * 16, 16)
                        gb[e, sl] = gb[e, sl] * w
                    return cc
                lax.fori_loop(0, bg, _scale, 0)
                db, ds_ = dstb.at[b], dsts.at[b]
                for j in range(bg // 16):
                    sl = pl.ds(j * 16, 16)
                    ds_[sl] = db[sl]
                scat_start(b)
                @pl.when(g + 2 < cpt)
                def _():
                    idx_start(g + 2, b)
            return carry
        lax.fori_loop(0, cpt // 2, _pair, 0)
        scat_wait(0)
        scat_wait(1)
        plsc.subcore_barrier()
        pltpu.sync_copy(accum.at[rows], out_hbm.at[c, rows])

    def _call(table, src, dst, ew, zeros):
        assert table.shape == tshape
        return _edge_pass(table, src, dst, ew, zeros)
    return _call


_edge_pass_split_feat = _make_edge_pass(n_tables=2, split_edges=False)
_edge_pass_split_edge = _make_edge_pass(n_tables=1, split_edges=True)


def _row_scale(cnt_col, i):
    row = lax.broadcasted_iota(jnp.int32, (_R,), 0) + i * _R
    deg = cnt_col - jnp.where(row == 0, jnp.float32(_PAD), jnp.float32(0.0))
    return lax.rsqrt(jnp.maximum(deg, 1.0))


def _dense1_body(x_ref, w_ref, cnt_ref, o_ref):
    i = pl.program_id(1)
    r_out = _row_scale(cnt_ref[0, :, 0], i)
    o_ref[0, :, :] = jnp.dot(x_ref[...], w_ref[...],
                             preferred_element_type=jnp.float32) * r_out[:, None]


def _dense1(x, w1, cnt):
    return pl.pallas_call(
        _dense1_body,
        grid=(2, _N // _R),
        in_specs=[
            pl.BlockSpec((_R, _D), lambda h, i: (i, 0)),
            pl.BlockSpec((_D, 128), lambda h, i: (0, h)),
            pl.BlockSpec((1, _R, 128), lambda h, i: (0, i, 0)),
        ],
        out_specs=pl.BlockSpec((1, _R, 128), lambda h, i: (h, i, 0)),
        out_shape=jax.ShapeDtypeStruct((2, _N, 128), jnp.float32),
    )(x, w1, cnt)


def _dense2_body(u_ref, cnt_ref, b1_ref, w2_ref, o_ref):
    i = pl.program_id(0)
    r_in = _row_scale(cnt_ref[1, :, 0], i)
    r_out = _row_scale(cnt_ref[0, :, 0], i)
    b1 = b1_ref[...]
    x1a = u_ref[0] * r_in[:, None] + b1[:, :128]
    x1b = u_ref[1] * r_in[:, None] + b1[:, 128:]
    h2 = (jnp.dot(x1a, w2_ref[:128, :], preferred_element_type=jnp.float32)
          + jnp.dot(x1b, w2_ref[128:, :], preferred_element_type=jnp.float32))
    o_ref[...] = h2 * r_out[:, None]


def _dense2(u1, cnt, b1, w2):
    return pl.pallas_call(
        _dense2_body,
        grid=(_N // _R,),
        in_specs=[
            pl.BlockSpec((2, _R, 128), lambda i: (0, i, 0)),
            pl.BlockSpec((2, _R, 128), lambda i: (0, i, 0)),
            pl.BlockSpec((1, _HID), lambda i: (0, 0)),
            pl.BlockSpec((_HID, _ND), lambda i: (0, 0)),
        ],
        out_specs=pl.BlockSpec((_R, _ND), lambda i: (i, 0)),
        out_shape=jax.ShapeDtypeStruct((_N, _ND), jnp.float32),
    )(u1, cnt, b1, w2)


def _dense3_body(u_ref, cnt_ref, w3_ref, b3_ref, o_ref):
    i = pl.program_id(0)
    r_in = _row_scale(cnt_ref[1, :, 0], i)
    x = jnp.maximum((u_ref[0] + u_ref[1]) * r_in[:, None], 0.0)
    logits = jnp.dot(x, w3_ref[...], preferred_element_type=jnp.float32) + b3_ref[...]
    m = jnp.max(logits, axis=1, keepdims=True)
    ex = jnp.exp(logits - m)
    o_ref[...] = ex / jnp.sum(ex, axis=1, keepdims=True)


def _dense3(u2, cnt, w3, b3):
    return pl.pallas_call(
        _dense3_body,
        grid=(_N // _R,),
        in_specs=[
            pl.BlockSpec((2, _R, 128), lambda i: (0, i, 0)),
            pl.BlockSpec((2, _R, 128), lambda i: (0, i, 0)),
            pl.BlockSpec((_ND, _NC), lambda i: (0, 0)),
            pl.BlockSpec((1, _NC), lambda i: (0, 0)),
        ],
        out_specs=pl.BlockSpec((_R, _NC), lambda i: (i, 0)),
        out_shape=jax.ShapeDtypeStruct((_N, _NC), jnp.float32),
    )(u2, cnt, w3, b3)


def kernel(features, edge_index, weight, edge_weight, W1, b1, W3, b3):
    idx2 = jnp.pad(edge_index, ((0, 0), (0, _PAD)))
    idx_flat = idx2.reshape(2 * _E_PAD)
    ewp = jnp.broadcast_to(jnp.pad(edge_weight, (0, _PAD))[:, None],
                           (_E_PAD, 16)).astype(jnp.float32)
    ones128 = jnp.ones((_B, 128), jnp.float32)
    z128 = jnp.zeros((_N_PAD, 128), jnp.float32)

    cnt = _degree_hist(idx_flat, ones128, z128)
    h1s = _dense1(features, W1, cnt)
    u1 = _edge_pass_split_feat(h1s, idx2[0], idx2[1], ewp, z128)
    h2s = _dense2(u1, cnt, b1.reshape(1, _HID), weight)
    u2 = _edge_pass_split_edge(h2s, idx2[0], idx2[1], ewp, z128)
    return _dense3(u2, cnt, W3, b3.reshape(1, _NC))


# speedup vs baseline: 3.9591x; 1.0411x over previous
import functools

import jax
import jax.numpy as jnp
from jax import lax
from jax.experimental import pallas as pl
from jax.experimental.pallas import tpu as pltpu
from jax.experimental.pallas import tpu_sc as plsc

_N = 10000
_E = 320000
_D = 128
_HID = 256
_ND = 128
_NC = 64

_B = 128
_E_PAD = 327680
_PAD = _E_PAD - _E
_N_SUB = 16
_N_PAD = 10112
_ROWS_PT = _N_PAD // _N_SUB
_R = 1000

_MESH = plsc.VectorSubcoreMesh(core_axis_name="c", subcore_axis_name="s")


@functools.partial(
    pl.kernel,
    out_type=jax.ShapeDtypeStruct((2, _N_PAD, 128), jnp.float32),
    mesh=_MESH,
    scratch_types=[
        pltpu.VMEM((2, _B), jnp.int32),
        pltpu.VMEM((2, _B), jnp.int32),
        pltpu.VMEM((_B, 128), jnp.float32),
        pltpu.VMEM_SHARED((_N_PAD, 128), jnp.float32),
        pltpu.SemaphoreType.DMA,
        pltpu.SemaphoreType.DMA,
        pltpu.SemaphoreType.DMA,
        pltpu.SemaphoreType.DMA,
    ],
)
def _degree_hist(idx_hbm, ones_hbm, zeros_hbm, out_hbm, idxb, idxs, onesb,
                 accum, si0, si1, ss0, ss1):
    c = lax.axis_index("c")
    s = lax.axis_index("s")
    rows = pl.ds(s * _ROWS_PT, _ROWS_PT)
    pltpu.sync_copy(ones_hbm, onesb)
    pltpu.sync_copy(zeros_hbm.at[rows], accum.at[rows])
    plsc.subcore_barrier()
    cpt = _E_PAD // _N_SUB // _B
    semi, sems = (si0, si1), (ss0, ss1)

    def idx_start(g, b):
        base = c * _E_PAD + (s * cpt + g) * _B
        pltpu.async_copy(idx_hbm.at[pl.ds(base, _B)], idxb.at[b], semi[b])

    def idx_wait(b):
        pltpu.make_async_copy(idx_hbm.at[pl.ds(0, _B)], idxb.at[b], semi[b]).wait()

    def scat_start(b):
        pltpu.async_copy(onesb, accum.at[idxs.at[b]], sems[b], add=True)

    def scat_wait(b):
        pltpu.make_async_copy(onesb, accum.at[idxs.at[b]], sems[b]).wait()

    idx_start(0, 0)
    idx_start(1, 1)

    def _pair(h, carry):
        for b in (0, 1):
            g = h * 2 + b
            idx_wait(b)
            @pl.when(g >= 2)
            def _():
                scat_wait(b)
            db, ds_ = idxb.at[b], idxs.at[b]
            for j in range(_B // 16):
                sl = pl.ds(j * 16, 16)
                ds_[sl] = db[sl]
            @pl.when(g + 2 < cpt)
            def _():
                idx_start(g + 2, b)
            scat_start(b)
        return carry
    lax.fori_loop(0, cpt // 2, _pair, 0)
    scat_wait(0)
    scat_wait(1)
    plsc.subcore_barrier()
    pltpu.sync_copy(accum.at[rows], out_hbm.at[c, rows])


def _make_edge_pass(n_tables, split_edges):
    n_workers = _N_SUB * (2 if split_edges else 1)
    bg = 64
    cpt = _E_PAD // n_workers // bg
    tshape = (2, _N, 128) if n_tables == 2 else (_N, 128)

    @functools.partial(
        pl.kernel,
        out_type=jax.ShapeDtypeStruct((2, _N_PAD, 128), jnp.float32),
        mesh=_MESH,
        scratch_types=[
            pltpu.VMEM((2, bg), jnp.int32),
            pltpu.VMEM((2, bg), jnp.int32),
            pltpu.VMEM((2, bg), jnp.int32),
            pltpu.VMEM((2, bg, 16), jnp.float32),
            pltpu.VMEM((2, bg, 128), jnp.float32),
            pltpu.VMEM_SHARED((_N_PAD, 128), jnp.float32),
            pltpu.SemaphoreType.DMA,
            pltpu.SemaphoreType.DMA,
            pltpu.SemaphoreType.DMA,
            pltpu.SemaphoreType.DMA,
            pltpu.SemaphoreType.DMA,
            pltpu.SemaphoreType.DMA,
        ],
    )
    def _edge_pass(table_hbm, src_hbm, dst_hbm, ew_hbm, zeros_hbm, out_hbm,
                   srcb, dstb, dsts, ewb, gbuf, accum,
                   si0, si1, sg0, sg1, ss0, ss1):
        c = lax.axis_index("c")
        s = lax.axis_index("s")
        rows = pl.ds(s * _ROWS_PT, _ROWS_PT)
        pltpu.sync_copy(zeros_hbm.at[rows], accum.at[rows])
        plsc.subcore_barrier()
        tbl = table_hbm.at[c] if n_tables == 2 else table_hbm
        wid = (s * 2 + c) if split_edges else s
        semi, semg, sems = (si0, si1), (sg0, sg1), (ss0, ss1)

        def idx_start(g, b):
            base = (wid * cpt + g) * bg
            pltpu.async_copy(src_hbm.at[pl.ds(base, bg)], srcb.at[b], semi[b])
            pltpu.async_copy(dst_hbm.at[pl.ds(base, bg)], dstb.at[b], semi[b])
            pltpu.async_copy(ew_hbm.at[pl.ds(base, bg)], ewb.at[b], semi[b])

        def idx_wait(b):
            pltpu.make_async_copy(src_hbm.at[pl.ds(0, bg)], srcb.at[b], semi[b]).wait()
            pltpu.make_async_copy(dst_hbm.at[pl.ds(0, bg)], dstb.at[b], semi[b]).wait()
            pltpu.make_async_copy(ew_hbm.at[pl.ds(0, bg)], ewb.at[b], semi[b]).wait()

        def gather_start(b):
            pltpu.async_copy(tbl.at[srcb.at[b]], gbuf.at[b], semg[b])

        def gather_wait(b):
            pltpu.make_async_copy(tbl.at[srcb.at[b]], gbuf.at[b], semg[b]).wait()

        def scat_start(b):
            pltpu.async_copy(gbuf.at[b], accum.at[dsts.at[b]], sems[b], add=True)

        def scat_wait(b):
            pltpu.make_async_copy(gbuf.at[b], accum.at[dsts.at[b]], sems[b]).wait()

        idx_start(0, 0)
        idx_wait(0)
        gather_start(0)
        idx_start(1, 1)

        def _pair(h, carry):
            for b in (0, 1):
                g = h * 2 + b
                nb = 1 - b
                @pl.when((g >= 1) & (g + 1 < cpt))
                def _():
                    scat_wait(nb)
                @pl.when(g + 1 < cpt)
                def _():
                    idx_wait(nb)
                    gather_start(nb)
                gather_wait(b)
                eb, gb = ewb.at[b], gbuf.at[b]
                def _scale(q, cc):
                    for d in range(4):
                        e = q * 4 + d
                        w = eb[e, :]
                        for j in range(8):
                            sl = pl.ds(j * 16, 16)
                            gb[e, sl] = gb[e, sl] * w
                    return cc
                lax.fori_loop(0, bg // 4, _scale, 0)
                db, ds_ = dstb.at[b], dsts.at[b]
                for j in range(bg // 16):
                    sl = pl.ds(j * 16, 16)
                    ds_[sl] = db[sl]
                scat_start(b)
                @pl.when(g + 2 < cpt)
                def _():
                    idx_start(g + 2, b)
            return carry
        lax.fori_loop(0, cpt // 2, _pair, 0)
        scat_wait(0)
        scat_wait(1)
        plsc.subcore_barrier()
        pltpu.sync_copy(accum.at[rows], out_hbm.at[c, rows])

    def _call(table, src, dst, ew, zeros):
        assert table.shape == tshape
        return _edge_pass(table, src, dst, ew, zeros)
    return _call


_edge_pass_split_feat = _make_edge_pass(n_tables=2, split_edges=False)
_edge_pass_split_edge = _make_edge_pass(n_tables=1, split_edges=True)


def _row_scale(cnt_col, i):
    row = lax.broadcasted_iota(jnp.int32, (_R,), 0) + i * _R
    deg = cnt_col - jnp.where(row == 0, jnp.float32(_PAD), jnp.float32(0.0))
    return lax.rsqrt(jnp.maximum(deg, 1.0))


def _dense1_body(x_ref, w_ref, cnt_ref, o_ref):
    i = pl.program_id(1)
    r_out = _row_scale(cnt_ref[0, :, 0], i)
    o_ref[0, :, :] = jnp.dot(x_ref[...], w_ref[...],
                             preferred_element_type=jnp.float32) * r_out[:, None]


def _dense1(x, w1, cnt):
    return pl.pallas_call(
        _dense1_body,
        grid=(2, _N // _R),
        in_specs=[
            pl.BlockSpec((_R, _D), lambda h, i: (i, 0)),
            pl.BlockSpec((_D, 128), lambda h, i: (0, h)),
            pl.BlockSpec((1, _R, 128), lambda h, i: (0, i, 0)),
        ],
        out_specs=pl.BlockSpec((1, _R, 128), lambda h, i: (h, i, 0)),
        out_shape=jax.ShapeDtypeStruct((2, _N, 128), jnp.float32),
    )(x, w1, cnt)


def _dense2_body(u_ref, cnt_ref, b1_ref, w2_ref, o_ref):
    i = pl.program_id(0)
    r_in = _row_scale(cnt_ref[1, :, 0], i)
    r_out = _row_scale(cnt_ref[0, :, 0], i)
    b1 = b1_ref[...]
    x1a = u_ref[0] * r_in[:, None] + b1[:, :128]
    x1b = u_ref[1] * r_in[:, None] + b1[:, 128:]
    h2 = (jnp.dot(x1a, w2_ref[:128, :], preferred_element_type=jnp.float32)
          + jnp.dot(x1b, w2_ref[128:, :], preferred_element_type=jnp.float32))
    o_ref[...] = h2 * r_out[:, None]


def _dense2(u1, cnt, b1, w2):
    return pl.pallas_call(
        _dense2_body,
        grid=(_N // _R,),
        in_specs=[
            pl.BlockSpec((2, _R, 128), lambda i: (0, i, 0)),
            pl.BlockSpec((2, _R, 128), lambda i: (0, i, 0)),
            pl.BlockSpec((1, _HID), lambda i: (0, 0)),
            pl.BlockSpec((_HID, _ND), lambda i: (0, 0)),
        ],
        out_specs=pl.BlockSpec((_R, _ND), lambda i: (i, 0)),
        out_shape=jax.ShapeDtypeStruct((_N, _ND), jnp.float32),
    )(u1, cnt, b1, w2)


def _dense3_body(u_ref, cnt_ref, w3_ref, b3_ref, o_ref):
    i = pl.program_id(0)
    r_in = _row_scale(cnt_ref[1, :, 0], i)
    x = jnp.maximum((u_ref[0] + u_ref[1]) * r_in[:, None], 0.0)
    logits = jnp.dot(x, w3_ref[...], preferred_element_type=jnp.float32) + b3_ref[...]
    m = jnp.max(logits, axis=1, keepdims=True)
    ex = jnp.exp(logits - m)
    o_ref[...] = ex / jnp.sum(ex, axis=1, keepdims=True)


def _dense3(u2, cnt, w3, b3):
    return pl.pallas_call(
        _dense3_body,
        grid=(_N // _R,),
        in_specs=[
            pl.BlockSpec((2, _R, 128), lambda i: (0, i, 0)),
            pl.BlockSpec((2, _R, 128), lambda i: (0, i, 0)),
            pl.BlockSpec((_ND, _NC), lambda i: (0, 0)),
            pl.BlockSpec((1, _NC), lambda i: (0, 0)),
        ],
        out_specs=pl.BlockSpec((_R, _NC), lambda i: (i, 0)),
        out_shape=jax.ShapeDtypeStruct((_N, _NC), jnp.float32),
    )(u2, cnt, w3, b3)


def kernel(features, edge_index, weight, edge_weight, W1, b1, W3, b3):
    idx2 = jnp.pad(edge_index, ((0, 0), (0, _PAD)))
    idx_flat = idx2.reshape(2 * _E_PAD)
    ewp = jnp.broadcast_to(jnp.pad(edge_weight, (0, _PAD))[:, None],
                           (_E_PAD, 16)).astype(jnp.float32)
    ones128 = jnp.ones((_B, 128), jnp.float32)
    z128 = jnp.zeros((_N_PAD, 128), jnp.float32)

    cnt = _degree_hist(idx_flat, ones128, z128)
    h1s = _dense1(features, W1, cnt)
    u1 = _edge_pass_split_feat(h1s, idx2[0], idx2[1], ewp, z128)
    h2s = _dense2(u1, cnt, b1.reshape(1, _HID), weight)
    u2 = _edge_pass_split_edge(h2s, idx2[0], idx2[1], ewp, z128)
    return _dense3(u2, cnt, W3, b3.reshape(1, _NC))
